# baseline (speedup 1.0000x reference)
import functools

import jax
import jax.numpy as jnp
from jax.experimental import pallas as pl
from jax.experimental.pallas import tpu as pltpu

_K = 16


def _knn_block_kernel(q_ref, k_ref, idx_ref, dist_ref, *, n_keys):
    q = q_ref[0]
    keys = k_ref[0]

    gram = jax.lax.dot_general(
        keys, q, (((0,), (0,)), ((), ())),
        preferred_element_type=jnp.float32)

    qn = jnp.sum(q * q, axis=0)
    kn = jnp.sum(keys * keys, axis=0)
    d2 = (kn[:, None] + qn[None, :]) - 2.0 * gram
    vals = jnp.maximum(d2, 0.0)

    r = vals.shape[1]
    quart = n_keys // 4
    iq = jax.lax.broadcasted_iota(jnp.int32, (quart, r), 0)
    grp = [(vals[j * quart:(j + 1) * quart, :], iq + j * quart)
           for j in range(4)]

    def ce(x, y, exact):
        (vx, ix), (vy, iy) = x, y
        if exact:
            keep = (vx < vy) | ((vx == vy) & (ix < iy))
        else:
            keep = vx <= vy
        return ((jnp.where(keep, vx, vy), jnp.where(keep, ix, iy)),
                (jnp.where(keep, vy, vx), jnp.where(keep, iy, ix)))

    grp[0], grp[1] = ce(grp[0], grp[1], False)
    grp[2], grp[3] = ce(grp[2], grp[3], False)
    grp[0], grp[2] = ce(grp[0], grp[2], False)
    grp[1], grp[3] = ce(grp[1], grp[3], False)
    grp[1], grp[2] = ce(grp[1], grp[2], True)
    (v0, i0), (v1, i1), (v2, i2), (v3, i3) = grp

    inf = jnp.float32(jnp.inf)
    for kk in range(_K):
        mv = jnp.min(v0, axis=0, keepdims=True)
        idx = jnp.min(jnp.where(v0 == mv, i0, n_keys), axis=0,
                      keepdims=True)
        dist_ref[0, kk, :] = mv[0, :]
        idx_ref[0, kk, :] = idx[0, :]
        wm = i0 == idx
        v0 = jnp.where(wm, v1, v0)
        i0 = jnp.where(wm, i1, i0)
        v1 = jnp.where(wm, v2, v1)
        i1 = jnp.where(wm, i2, i1)
        v2 = jnp.where(wm, v3, v2)
        i2 = jnp.where(wm, i3, i2)
        v3 = jnp.where(wm, inf, v3)
        i3 = jnp.where(wm, n_keys, i3)


def kernel(positions):
    b, d, n = positions.shape
    r = 256
    grid = (b, n // r)
    fn = functools.partial(_knn_block_kernel, n_keys=n)
    idx, dist = pl.pallas_call(
        fn,
        grid=grid,
        in_specs=[
            pl.BlockSpec((1, d, r), lambda bi, ri: (bi, 0, ri)),
            pl.BlockSpec((1, d, n), lambda bi, ri: (bi, 0, 0)),
        ],
        out_specs=[
            pl.BlockSpec((1, _K, r), lambda bi, ri: (bi, 0, ri)),
            pl.BlockSpec((1, _K, r), lambda bi, ri: (bi, 0, ri)),
        ],
        out_shape=[
            jax.ShapeDtypeStruct((b, _K, n), jnp.int32),
            jax.ShapeDtypeStruct((b, _K, n), jnp.float32),
        ],
        compiler_params=pltpu.CompilerParams(
            dimension_semantics=("parallel", "parallel")),
    )(positions, positions)
    return idx, dist


# speedup vs baseline: 1.2341x; 1.2341x over previous
import functools

import jax
import jax.numpy as jnp
from jax.experimental import pallas as pl
from jax.experimental.pallas import tpu as pltpu

_K = 16


def _knn_block_kernel(q_ref, k_ref, idx_ref, dist_ref, *, n_keys):
    q = q_ref[0]
    keys = k_ref[0]

    gram = jax.lax.dot_general(
        keys, q, (((0,), (0,)), ((), ())),
        preferred_element_type=jnp.float32)

    qn = jnp.sum(q * q, axis=0)
    kn = jnp.sum(keys * keys, axis=0)
    d2 = (kn[:, None] + qn[None, :]) - 2.0 * gram
    vals = jnp.maximum(d2, 0.0)

    r = vals.shape[1]
    quart = n_keys // 4
    iq = jax.lax.broadcasted_iota(jnp.int32, (quart, r), 0)
    grp = [(vals[j * quart:(j + 1) * quart, :], iq + j * quart)
           for j in range(4)]

    def ce(x, y, exact):
        (vx, ix), (vy, iy) = x, y
        if exact:
            keep = (vx < vy) | ((vx == vy) & (ix < iy))
        else:
            keep = vx <= vy
        return ((jnp.where(keep, vx, vy), jnp.where(keep, ix, iy)),
                (jnp.where(keep, vy, vx), jnp.where(keep, iy, ix)))

    grp[0], grp[1] = ce(grp[0], grp[1], False)
    grp[2], grp[3] = ce(grp[2], grp[3], False)
    grp[0], grp[2] = ce(grp[0], grp[2], False)
    grp[1], grp[3] = ce(grp[1], grp[3], False)
    grp[1], grp[2] = ce(grp[1], grp[2], True)
    (v0, i0), (v1, i1), (v2, i2), (v3, i3) = grp

    inf = jnp.float32(jnp.inf)
    for kk in range(_K):
        mv = jnp.min(v0, axis=0, keepdims=True)
        idx = jnp.min(jnp.where(v0 == mv, i0, n_keys), axis=0,
                      keepdims=True)
        dist_ref[0, kk, :] = mv[0, :]
        idx_ref[0, kk, :] = idx[0, :]
        wm = i0 == idx
        v0 = jnp.where(wm, v1, v0)
        i0 = jnp.where(wm, i1, i0)
        v1 = jnp.where(wm, v2, v1)
        i1 = jnp.where(wm, i2, i1)
        v2 = jnp.where(wm, v3, v2)
        i2 = jnp.where(wm, i3, i2)
        v3 = jnp.where(wm, inf, v3)


def kernel(positions):
    b, d, n = positions.shape
    r = 512
    grid = (b, n // r)
    fn = functools.partial(_knn_block_kernel, n_keys=n)
    idx, dist = pl.pallas_call(
        fn,
        grid=grid,
        in_specs=[
            pl.BlockSpec((1, d, r), lambda bi, ri: (bi, 0, ri)),
            pl.BlockSpec((1, d, n), lambda bi, ri: (bi, 0, 0)),
        ],
        out_specs=[
            pl.BlockSpec((1, _K, r), lambda bi, ri: (bi, 0, ri)),
            pl.BlockSpec((1, _K, r), lambda bi, ri: (bi, 0, ri)),
        ],
        out_shape=[
            jax.ShapeDtypeStruct((b, _K, n), jnp.int32),
            jax.ShapeDtypeStruct((b, _K, n), jnp.float32),
        ],
        compiler_params=pltpu.CompilerParams(
            dimension_semantics=("parallel", "parallel")),
    )(positions, positions)
    return idx, dist
